# Initial kernel scaffold; baseline (speedup 1.0000x reference)
import jax
import jax.numpy as jnp
from jax.experimental import pallas as pl


def kernel(x, weights, indices, W1, W2):
    raise NotImplementedError("write your pallas kernel here")


# speedup vs baseline: 2.4096x; 2.4096x over previous
import functools

import jax
import jax.numpy as jnp
from jax import lax
from jax.experimental import pallas as pl
from jax.experimental.pallas import tpu as pltpu
from jax.experimental.pallas import tpu_sc as plsc

E = 16
T = 2048
D = 1024
H = 1024
K = 2
N = T * K
BT = 256
NB = N // BT
MAXT = NB + E - 1

_NC = 2
_NS = 16
_NW = _NC * _NS


def _routing_metadata(weights, indices):
    eid = indices.reshape(-1).astype(jnp.int32)
    order = jnp.argsort(eid, stable=True).astype(jnp.int32)
    tok = (order // K).astype(jnp.int32)
    sw = weights.reshape(-1)[order]
    inv = jnp.zeros((N,), jnp.int32).at[order].set(
        jnp.arange(N, dtype=jnp.int32))
    counts = jnp.zeros((E,), jnp.int32).at[eid].add(1)
    offs = jnp.concatenate(
        [jnp.zeros((1,), jnp.int32), jnp.cumsum(counts).astype(jnp.int32)])
    fb = offs[:E] // BT
    lb = jnp.maximum(offs[1:] - 1, 0) // BT
    tiles_per = jnp.where(counts > 0, lb - fb + 1, 0).astype(jnp.int32)
    toffs = jnp.concatenate(
        [jnp.zeros((1,), jnp.int32), jnp.cumsum(tiles_per).astype(jnp.int32)])
    total = toffs[E]
    ti = jnp.arange(MAXT, dtype=jnp.int32)
    e_raw = jnp.clip(
        jnp.searchsorted(toffs, ti, side="right").astype(jnp.int32) - 1,
        0, E - 1)
    b_raw = fb[e_raw] + (ti - toffs[e_raw])
    valid = ti < total
    e_last = jnp.max(jnp.where(counts > 0, jnp.arange(E, dtype=jnp.int32), -1))
    eid_t = jnp.where(valid, e_raw, e_last).astype(jnp.int32)
    blk_t = jnp.where(valid, b_raw, NB - 1).astype(jnp.int32)
    lo_g = jnp.maximum(offs[eid_t], blk_t * BT)
    hi_g = jnp.minimum(offs[eid_t + 1], (blk_t + 1) * BT)
    lo_t = jnp.where(valid, lo_g - blk_t * BT, 0).astype(jnp.int32)
    hi_t = jnp.where(valid, hi_g - blk_t * BT, 0).astype(jnp.int32)
    first_t = jnp.concatenate(
        [jnp.ones((1,), jnp.int32),
         (blk_t[1:] != blk_t[:-1]).astype(jnp.int32)])
    return tok, sw, inv, blk_t, eid_t, lo_t, hi_t, first_t


def _sc_gather(table, idx):
    B = idx.shape[0]
    Dd = table.shape[1]
    bpw = B // _NW
    ch = 32
    nch = bpw // ch
    mesh = plsc.VectorSubcoreMesh(core_axis_name="c", subcore_axis_name="s")

    @functools.partial(
        pl.kernel,
        mesh=mesh,
        out_type=jax.ShapeDtypeStruct((B, Dd), table.dtype),
        scratch_types=[
            pltpu.VMEM((ch,), jnp.int32),
            pltpu.VMEM((ch, Dd), table.dtype),
            pltpu.SemaphoreType.DMA,
        ],
    )
    def k(table_hbm, idx_hbm, out_hbm, idx_v, buf, sem):
        wid = lax.axis_index("s") * _NC + lax.axis_index("c")
        base = wid * bpw

        def body(c, carry):
            off = base + c * ch
            pltpu.sync_copy(idx_hbm.at[pl.ds(off, ch)], idx_v)
            pltpu.async_copy(table_hbm.at[idx_v], buf, sem).wait()
            pltpu.sync_copy(buf, out_hbm.at[pl.ds(off, ch)])
            return carry

        lax.fori_loop(0, nch, body, 0)

    return k(table, idx)


def _gmm_body(blk_r, eid_r, lo_r, hi_r, first_r,
              xs_r, w1_r, w2_r, swb_r, out_r):
    i = pl.program_id(0)
    lo = lo_r[i]
    hi = hi_r[i]
    rows = lax.broadcasted_iota(jnp.int32, (BT, 1), 0)
    mask = (rows >= lo) & (rows < hi)
    xb = jnp.where(mask, xs_r[...], 0.0)
    h = jnp.dot(xb, w1_r[0], preferred_element_type=jnp.float32)
    yv = h[:, :H]
    g = h[:, H:]
    act = yv * (g * jax.nn.sigmoid(g))
    o = jnp.dot(act, w2_r[0], preferred_element_type=jnp.float32)
    o = o * swb_r[:, :1]

    @pl.when(first_r[i] != 0)
    def _():
        out_r[...] = o

    @pl.when(first_r[i] == 0)
    def _():
        out_r[...] = out_r[...] + o


def _grouped_mlp(xs, W1, W2, swb, blk_t, eid_t, lo_t, hi_t, first_t):
    grid_spec = pltpu.PrefetchScalarGridSpec(
        num_scalar_prefetch=5,
        grid=(MAXT,),
        in_specs=[
            pl.BlockSpec((BT, D), lambda i, b, e, l, h, f: (b[i], 0)),
            pl.BlockSpec((1, D, 2 * H), lambda i, b, e, l, h, f: (e[i], 0, 0)),
            pl.BlockSpec((1, H, D), lambda i, b, e, l, h, f: (e[i], 0, 0)),
            pl.BlockSpec((BT, 128), lambda i, b, e, l, h, f: (b[i], 0)),
        ],
        out_specs=pl.BlockSpec((BT, D), lambda i, b, e, l, h, f: (b[i], 0)),
    )
    return pl.pallas_call(
        _gmm_body,
        grid_spec=grid_spec,
        out_shape=jax.ShapeDtypeStruct((N, D), jnp.float32),
    )(blk_t, eid_t, lo_t, hi_t, first_t, xs, W1, W2, swb)


def _pair_add(yp2):
    bt = 256

    def body(a_r, o_r):
        o_r[...] = a_r[:, :D] + a_r[:, D:]

    return pl.pallas_call(
        body,
        grid=(T // bt,),
        in_specs=[pl.BlockSpec((bt, 2 * D), lambda i: (i, 0))],
        out_specs=pl.BlockSpec((bt, D), lambda i: (i, 0)),
        out_shape=jax.ShapeDtypeStruct((T, D), jnp.float32),
    )(yp2)


def kernel(x, weights, indices, W1, W2):
    tok, sw, inv, blk_t, eid_t, lo_t, hi_t, first_t = _routing_metadata(
        weights, indices)
    swb = jnp.broadcast_to(sw[:, None], (N, 128))
    xs = _sc_gather(x, tok)
    ysw = _grouped_mlp(xs, W1, W2, swb, blk_t, eid_t, lo_t, hi_t, first_t)
    yp = _sc_gather(ysw, inv)
    return _pair_add(yp.reshape(T, 2 * D))


# speedup vs baseline: 2.6089x; 1.0827x over previous
import functools

import jax
import jax.numpy as jnp
from jax import lax
from jax.experimental import pallas as pl
from jax.experimental.pallas import tpu as pltpu
from jax.experimental.pallas import tpu_sc as plsc

E = 16
T = 2048
D = 1024
H = 1024
K = 2
N = T * K
BT = 256
NB = N // BT
MAXT = NB + E - 1

_NC = 2
_NS = 16
_NW = _NC * _NS


def _routing_metadata(weights, indices):
    eid = indices.reshape(-1).astype(jnp.int32)
    order = jnp.argsort(eid, stable=True).astype(jnp.int32)
    tok = (order // K).astype(jnp.int32)
    sw = weights.reshape(-1)[order]
    inv = jnp.argsort(order).astype(jnp.int32)
    sorted_eid = eid[order]
    offs = jnp.searchsorted(
        sorted_eid, jnp.arange(E + 1, dtype=jnp.int32), side="left"
    ).astype(jnp.int32)
    counts = offs[1:] - offs[:E]
    fb = offs[:E] // BT
    lb = jnp.maximum(offs[1:] - 1, 0) // BT
    tiles_per = jnp.where(counts > 0, lb - fb + 1, 0).astype(jnp.int32)
    toffs = jnp.concatenate(
        [jnp.zeros((1,), jnp.int32), jnp.cumsum(tiles_per).astype(jnp.int32)])
    total = toffs[E]
    ti = jnp.arange(MAXT, dtype=jnp.int32)
    e_raw = jnp.clip(
        jnp.searchsorted(toffs, ti, side="right").astype(jnp.int32) - 1,
        0, E - 1)
    b_raw = fb[e_raw] + (ti - toffs[e_raw])
    valid = ti < total
    e_last = jnp.max(jnp.where(counts > 0, jnp.arange(E, dtype=jnp.int32), -1))
    eid_t = jnp.where(valid, e_raw, e_last).astype(jnp.int32)
    blk_t = jnp.where(valid, b_raw, NB - 1).astype(jnp.int32)
    lo_g = jnp.maximum(offs[eid_t], blk_t * BT)
    hi_g = jnp.minimum(offs[eid_t + 1], (blk_t + 1) * BT)
    lo_t = jnp.where(valid, lo_g - blk_t * BT, 0).astype(jnp.int32)
    hi_t = jnp.where(valid, hi_g - blk_t * BT, 0).astype(jnp.int32)
    first_t = jnp.concatenate(
        [jnp.ones((1,), jnp.int32),
         (blk_t[1:] != blk_t[:-1]).astype(jnp.int32)])
    return tok, sw, inv, blk_t, eid_t, lo_t, hi_t, first_t


def _sc_gather(table, idx):
    B = idx.shape[0]
    Dd = table.shape[1]
    bpw = B // _NW
    ch = 32
    nch = bpw // ch
    mesh = plsc.VectorSubcoreMesh(core_axis_name="c", subcore_axis_name="s")

    @functools.partial(
        pl.kernel,
        mesh=mesh,
        out_type=jax.ShapeDtypeStruct((B, Dd), table.dtype),
        scratch_types=[
            pltpu.VMEM((bpw,), jnp.int32),
            pltpu.VMEM((ch, Dd), table.dtype),
            pltpu.VMEM((ch, Dd), table.dtype),
            pltpu.SemaphoreType.DMA,
            pltpu.SemaphoreType.DMA,
            pltpu.SemaphoreType.DMA,
            pltpu.SemaphoreType.DMA,
        ],
    )
    def k(table_hbm, idx_hbm, out_hbm, idx_v, buf0, buf1, sg0, sg1, sw0, sw1):
        wid = lax.axis_index("s") * _NC + lax.axis_index("c")
        base = wid * bpw
        bufs = (buf0, buf1)
        sgs = (sg0, sg1)
        sws = (sw0, sw1)
        pltpu.sync_copy(idx_hbm.at[pl.ds(base, bpw)], idx_v)
        writes = [None, None]
        for c in range(nch):
            s = c % 2
            if writes[s] is not None:
                writes[s].wait()
            pltpu.async_copy(
                table_hbm.at[idx_v.at[pl.ds(c * ch, ch)]], bufs[s], sgs[s]
            ).wait()
            writes[s] = pltpu.async_copy(
                bufs[s], out_hbm.at[pl.ds(base + c * ch, ch)], sws[s])
        writes[0].wait()
        writes[1].wait()

    return k(table, idx)


def _sc_pair_add(ysw, inv):
    tpw = T // _NW
    ct = 16
    nch = tpw // ct
    nvec = ct * (D // 16)
    mesh = plsc.VectorSubcoreMesh(core_axis_name="c", subcore_axis_name="s")

    @functools.partial(
        pl.kernel,
        mesh=mesh,
        out_type=jax.ShapeDtypeStruct((T, D), jnp.float32),
        scratch_types=[
            pltpu.VMEM((2 * tpw,), jnp.int32),
            pltpu.VMEM((2 * ct, D), jnp.float32),
            pltpu.VMEM((2 * ct, D), jnp.float32),
            pltpu.VMEM((ct, D), jnp.float32),
            pltpu.VMEM((ct, D), jnp.float32),
            pltpu.SemaphoreType.DMA,
            pltpu.SemaphoreType.DMA,
            pltpu.SemaphoreType.DMA,
            pltpu.SemaphoreType.DMA,
        ],
    )
    def k(ysw_hbm, inv_hbm, y_hbm, idx_v, buf0, buf1, ob0, ob1,
          sg0, sg1, sw0, sw1):
        wid = lax.axis_index("s") * _NC + lax.axis_index("c")
        tbase = wid * tpw
        bufs = (buf0, buf1)
        obs = (ob0, ob1)
        sgs = (sg0, sg1)
        sws = (sw0, sw1)
        pltpu.sync_copy(inv_hbm.at[pl.ds(2 * tbase, 2 * tpw)], idx_v)
        gathers = [None, None]
        writes = [None, None]
        gathers[0] = pltpu.async_copy(
            ysw_hbm.at[idx_v.at[pl.ds(0, 2 * ct)]], buf0, sg0)
        for c in range(nch):
            s = c % 2
            gathers[s].wait()
            if c + 1 < nch:
                s1 = (c + 1) % 2
                gathers[s1] = pltpu.async_copy(
                    ysw_hbm.at[idx_v.at[pl.ds((c + 1) * 2 * ct, 2 * ct)]],
                    bufs[s1], sgs[s1])
            if writes[s] is not None:
                writes[s].wait()
            buf = bufs[s]
            ob = obs[s]

            def body(i, carry, buf=buf, ob=ob):
                j = i // (D // 16)
                v = (i % (D // 16)) * 16
                ob[j, pl.ds(v, 16)] = (
                    buf[2 * j, pl.ds(v, 16)] + buf[2 * j + 1, pl.ds(v, 16)])
                return carry

            lax.fori_loop(0, nvec, body, 0)
            writes[s] = pltpu.async_copy(
                ob, y_hbm.at[pl.ds(tbase + c * ct, ct)], sws[s])
        writes[0].wait()
        writes[1].wait()

    return k(ysw, inv)


def _gmm_body(blk_r, eid_r, lo_r, hi_r, first_r,
              xs_r, w1_r, w2_r, swb_r, out_r):
    i = pl.program_id(0)
    lo = lo_r[i]
    hi = hi_r[i]
    rows = lax.broadcasted_iota(jnp.int32, (BT, 1), 0)
    mask = (rows >= lo) & (rows < hi)
    xb = jnp.where(mask, xs_r[...], 0.0)
    h = jnp.dot(xb, w1_r[0], preferred_element_type=jnp.float32)
    yv = h[:, :H]
    g = h[:, H:]
    act = yv * (g * jax.nn.sigmoid(g))
    o = jnp.dot(act, w2_r[0], preferred_element_type=jnp.float32)
    o = o * swb_r[:, :1]

    @pl.when(first_r[i] != 0)
    def _():
        out_r[...] = o

    @pl.when(first_r[i] == 0)
    def _():
        out_r[...] = out_r[...] + o


def _grouped_mlp(xs, W1, W2, swb, blk_t, eid_t, lo_t, hi_t, first_t):
    grid_spec = pltpu.PrefetchScalarGridSpec(
        num_scalar_prefetch=5,
        grid=(MAXT,),
        in_specs=[
            pl.BlockSpec((BT, D), lambda i, b, e, l, h, f: (b[i], 0)),
            pl.BlockSpec((1, D, 2 * H), lambda i, b, e, l, h, f: (e[i], 0, 0)),
            pl.BlockSpec((1, H, D), lambda i, b, e, l, h, f: (e[i], 0, 0)),
            pl.BlockSpec((BT, 128), lambda i, b, e, l, h, f: (b[i], 0)),
        ],
        out_specs=pl.BlockSpec((BT, D), lambda i, b, e, l, h, f: (b[i], 0)),
    )
    return pl.pallas_call(
        _gmm_body,
        grid_spec=grid_spec,
        out_shape=jax.ShapeDtypeStruct((N, D), jnp.float32),
    )(blk_t, eid_t, lo_t, hi_t, first_t, xs, W1, W2, swb)


def kernel(x, weights, indices, W1, W2):
    tok, sw, inv, blk_t, eid_t, lo_t, hi_t, first_t = _routing_metadata(
        weights, indices)
    swb = jnp.broadcast_to(sw[:, None], (N, 128))
    xs = _sc_gather(x, tok)
    ysw = _grouped_mlp(xs, W1, W2, swb, blk_t, eid_t, lo_t, hi_t, first_t)
    return _sc_pair_add(ysw, inv)
